# baseline (speedup 1.0000x reference)
import functools

import jax
import jax.numpy as jnp
from jax import lax
from jax.experimental import pallas as pl
from jax.experimental.pallas import tpu as pltpu
from jax.experimental.pallas import tpu_sc as plsc

VOCAB = 1000000
EMBED = 32
BATCH = 4096
SEQ = 200

ROW = 128
NROWS = BATCH * SEQ // ROW
NW = 32
ROWS_PER_W = NROWS // NW

NBLK = VOCAB // ROW
NTJ = NBLK + 1
VPAD = NTJ * ROW
LIN_ROWS = VOCAB * EMBED // ROW

_MESH = dict(core_axis_name="c", subcore_axis_name="s")
_LINEAR_PARAMS = pltpu.CompilerParams(
    use_tc_tiling_on_sc=False, needs_layout_passes=False
)


def _wid():
  return lax.axis_index("s") * 2 + lax.axis_index("c")


def _make_a1():
  @functools.partial(
      pl.kernel,
      mesh=plsc.VectorSubcoreMesh(**_MESH),
      compiler_params=pltpu.CompilerParams(use_tc_tiling_on_sc=True),
      out_type=jax.ShapeDtypeStruct((4, 8, VPAD), jnp.float32),
      scratch_types=[pltpu.SemaphoreType.DMA],
  )
  def body(tt_hbm, raw_hbm, sem):
    wid = _wid()
    et = wid // 8
    k = wid % 8
    W0 = 977 * ROW
    W1 = 976 * ROW
    CW = 122 * ROW
    x0 = jnp.where(k < 4, k * W0, 4 * W0 + (k - 4) * W1)
    copies = []
    for j in range(8):
      off = x0 + j * CW
      copies.append(pltpu.async_copy(
          tt_hbm.at[pl.ds(et * 8, 8), pl.ds(off, CW)],
          raw_hbm.at[et, :, pl.ds(off, CW)],
          sem,
      ))

    @pl.when(k < 4)
    def _():
      off = x0 + 8 * CW
      pltpu.async_copy(
          tt_hbm.at[pl.ds(et * 8, 8), pl.ds(off, ROW)],
          raw_hbm.at[et, :, pl.ds(off, ROW)],
          sem,
      ).wait()

    for cp in copies:
      cp.wait()

  return body


def _transpose_block(dst, src, src_rows, j0, dst_row0, n_vregs, splat3=None):
  iota = lax.iota(jnp.int32, 16)
  b_vecs = [iota + (h * 16) for h in range(8)]
  nd = len(dst.shape)
  for j in range(n_vregs):
    e0 = j // 8
    bi_vec = b_vecs[j % 8]
    if splat3 is None:
      v = plsc.load_gather(src, [bi_vec, jnp.full((16,), e0, jnp.int32)])
    else:
      v = plsc.load_gather(
          src,
          [
              jnp.full((16,), splat3 + e0 // 8, jnp.int32),
              jnp.full((16,), e0 % 8, jnp.int32),
              bi_vec,
          ],
      )
    flat = 16 * (j0 + j)
    if nd == 2:
      dst[dst_row0 + flat // 128, pl.ds(flat % 128, 16)] = v
    else:
      dst[flat // 1024, (flat % 1024) // 128, pl.ds(flat % 128, 16)] = v


def _make_a2():
  @functools.partial(
      pl.kernel,
      mesh=plsc.VectorSubcoreMesh(**_MESH),
      compiler_params=_LINEAR_PARAMS,
      out_type=jax.ShapeDtypeStruct((LIN_ROWS, ROW), jnp.float32),
      scratch_types=[
          pltpu.VMEM((8, 8, ROW), jnp.float32),
          pltpu.VMEM((8, 8, ROW), jnp.float32),
          pltpu.VMEM((64, ROW), jnp.float32),
          pltpu.VMEM((64, ROW), jnp.float32),
          pltpu.SemaphoreType.DMA,
          pltpu.SemaphoreType.DMA,
          pltpu.SemaphoreType.DMA,
          pltpu.SemaphoreType.DMA,
      ],
  )
  def body(raw_hbm, tail_hbm, lin_hbm, ina, inb, outa, outb,
           gsa, gsb, wsa, wsb):
    wid = _wid()
    ng = 61 + (wid == 0)
    c0 = jnp.where(wid == 0, 0, 248 + (wid - 1) * 244)

    def fire_in(g, buf, sem):
      c = c0 + 2 * g
      for et in range(4):
        pltpu.async_copy(raw_hbm.at[pl.ds(et * NTJ + c, 2)],
                         buf.at[pl.ds(et * 2, 2)], sem)

    def wait_in(buf, sem):
      for _ in range(4):
        pltpu.make_async_copy(raw_hbm.at[pl.ds(0, 2)],
                              buf.at[pl.ds(0, 2)], sem).wait()

    def transpose(buf, out):
      iota = lax.iota(jnp.int32, 16)
      e_vecs = [iota + h * 16 for h in range(2)]
      d1s = [e & 7 for e in e_vecs]
      for i in range(2):
        d0s = [(e >> 3) * 2 + i for e in e_vecs]
        for j in range(256):
          half = j % 2
          v = plsc.load_gather(
              buf,
              [d0s[half], d1s[half], jnp.full((16,), j // 2, jnp.int32)],
          )
          flat = 16 * j
          out[i * 32 + flat // 128, pl.ds(flat % 128, 16)] = v

    def fire_out(g, buf, sem):
      pltpu.async_copy(buf, lin_hbm.at[pl.ds((c0 + 2 * g) * EMBED, 64)], sem)

    def wait_out(buf, sem):
      pltpu.make_async_copy(buf, lin_hbm.at[pl.ds(0, 64)], sem).wait()

    fire_in(0, ina, gsa)
    fire_in(1, inb, gsb)

    def step(m, carry):
      ga = 2 * m
      gb = ga + 1
      wait_in(ina, gsa)

      @pl.when(m >= 1)
      def _():
        wait_out(outa, wsa)

      transpose(ina, outa)

      @pl.when(m < ng - 1)
      def _():
        fire_in(ga + 2, ina, gsa)

      fire_out(ga, outa, wsa)
      wait_in(inb, gsb)

      @pl.when(m >= 1)
      def _():
        wait_out(outb, wsb)

      transpose(inb, outb)

      @pl.when(m < ng - 1)
      def _():
        fire_in(gb + 2, inb, gsb)

      fire_out(gb, outb, wsb)
      return carry

    lax.fori_loop(0, ng, step, 0)
    wait_out(outa, wsa)
    wait_out(outb, wsb)

    @pl.when(wid == 17)
    def _tail():
      pltpu.sync_copy(tail_hbm, lin_hbm.at[pl.ds(NBLK * EMBED, 16)])

  return body


def _make_phase_b():
  @functools.partial(
      pl.kernel,
      mesh=plsc.VectorSubcoreMesh(**_MESH),
      compiler_params=_LINEAR_PARAMS,
      out_type=jax.ShapeDtypeStruct((NROWS * 4, 8, ROW), jnp.float32),
      scratch_types=(
          [pltpu.VMEM((ROWS_PER_W, ROW), jnp.int32)]
          + [pltpu.VMEM((ROW, EMBED), jnp.float32) for _ in range(4)]
          + [pltpu.VMEM((4, 8, ROW), jnp.float32) for _ in range(4)]
          + [pltpu.SemaphoreType.DMA for _ in range(8)]
      ),
  )
  def body(idx_hbm, tab_hbm, out_hbm, idx_all, *bufs):
    rows = bufs[0:4]
    trs = bufs[4:8]
    gs = bufs[8:12]
    ws = bufs[12:16]
    wid = _wid()
    q0 = wid * ROWS_PER_W
    pltpu.sync_copy(idx_hbm.at[pl.ds(q0, ROWS_PER_W)], idx_all)

    def fire_g(n, k):
      pltpu.async_copy(tab_hbm.at[idx_all.at[n]], rows[k], gs[k])

    def wait_g(k):
      pltpu.make_async_copy(tab_hbm.at[pl.ds(0, ROW)], rows[k], gs[k]).wait()

    def transpose(k):
      _transpose_block(trs[k], rows[k], None, j0=0, dst_row0=0, n_vregs=256)

    def fire_w(n, k):
      q = q0 + n
      st = q // 256
      r = q % 256
      bt = r // 8
      si = r % 8
      s = st * 8 + si
      for et in range(4):
        pltpu.async_copy(trs[k].at[et], out_hbm.at[(s * 4 + et) * 32 + bt],
                         ws[k])

    def wait_w(k):
      pltpu.make_async_copy(trs[k], out_hbm.at[pl.ds(0, 4)], ws[k]).wait()

    for k in range(4):
      fire_g(k, k)

    def step(t, carry):
      for k in range(4):
        n = 4 * t + k
        wait_g(k)

        @pl.when(t >= 1)
        def _(k=k):
          wait_w(k)

        transpose(k)

        @pl.when(t < 49)
        def _(n=n, k=k):
          fire_g(n + 4, k)

        fire_w(n, k)
      return carry

    lax.fori_loop(0, 50, step, 0)
    for k in range(4):
      wait_w(k)

  return body


_a1 = _make_a1()
_a2 = _make_a2()
_phase_b = _make_phase_b()


def kernel(words_seq, table):
  raw = _a1(table.T)
  raw3 = raw.reshape(4, 8, NTJ, ROW).transpose(0, 2, 1, 3).reshape(
      4 * NTJ, 8, ROW)
  tail = table[NBLK * ROW :, :].reshape(16, ROW)
  tab = _a2(raw3, tail).reshape(VOCAB, EMBED)
  ws = words_seq.astype(jnp.int32)
  idx = ws.T.reshape(25, 8, 32, 128).transpose(0, 2, 1, 3).reshape(NROWS, ROW)
  out = _phase_b(idx, tab)
  out5 = out.reshape(SEQ, 4, 32, 8, ROW)
  return out5.transpose(2, 4, 0, 1, 3).reshape(BATCH, SEQ, EMBED)


# speedup vs baseline: 3.4508x; 3.4508x over previous
import functools

import jax
import jax.numpy as jnp
from jax import lax
from jax.experimental import pallas as pl
from jax.experimental.pallas import tpu as pltpu
from jax.experimental.pallas import tpu_sc as plsc

VOCAB = 1000000
EMBED = 32
BATCH = 4096
SEQ = 200

ROW = 128
NROWS = BATCH * SEQ // ROW
NW = 32
ROWS_PER_W = NROWS // NW

NBLK = VOCAB // ROW
NTJ = NBLK + 1
VPAD = NTJ * ROW
LIN_ROWS = VOCAB * EMBED // ROW

_MESH = dict(core_axis_name="c", subcore_axis_name="s")
_LINEAR_PARAMS = pltpu.CompilerParams(
    use_tc_tiling_on_sc=False, needs_layout_passes=False
)


def _wid():
  return lax.axis_index("s") * 2 + lax.axis_index("c")


def _make_a1():
  CW = 8 * ROW
  NCH = 122

  @functools.partial(
      pl.kernel,
      mesh=plsc.VectorSubcoreMesh(**_MESH),
      compiler_params=pltpu.CompilerParams(use_tc_tiling_on_sc=True),
      out_type=jax.ShapeDtypeStruct((4, 8, VPAD), jnp.float32),
      scratch_types=(
          [pltpu.VMEM((8, CW), jnp.float32) for _ in range(8)]
          + [pltpu.SemaphoreType.DMA for _ in range(16)]
      ),
  )
  def body(tt_hbm, raw_hbm, *sc):
    bufs = sc[0:8]
    rs = sc[8:16]
    ws2 = sc[16:24]
    wid = _wid()
    et = wid // 8
    k = wid % 8
    W0 = 977 * ROW
    W1 = 976 * ROW
    x0 = jnp.where(k < 4, k * W0, 4 * W0 + (k - 4) * W1)

    def fire_in(t, b):
      off = x0 + t * CW
      pltpu.async_copy(tt_hbm.at[pl.ds(et * 8, 8), pl.ds(off, CW)],
                       bufs[b], rs[b])

    def wait_in(b):
      pltpu.make_async_copy(tt_hbm.at[pl.ds(0, 8), pl.ds(0, CW)],
                            bufs[b], rs[b]).wait()

    def fire_out(t, b):
      off = x0 + t * CW
      pltpu.async_copy(bufs[b], raw_hbm.at[et, :, pl.ds(off, CW)], ws2[b])

    def wait_out(b):
      pltpu.make_async_copy(bufs[b], raw_hbm.at[0, :, pl.ds(0, CW)],
                            ws2[b]).wait()

    for b in range(4):
      fire_in(b, b)

    def step(t, carry):
      for b in range(8):
        u = 8 * t + b

        @pl.when(u < NCH)
        def _(u=u, b=b):
          wait_in(b)
          fire_out(u, b)

          @pl.when(u >= 4)
          def _(u=u, b=b):
            wait_out((b + 4) % 8)

          @pl.when(u + 4 < NCH)
          def _(u=u, b=b):
            fire_in(u + 4, (b + 4) % 8)

      return carry

    lax.fori_loop(0, 16, step, 0)
    wait_out(6)
    wait_out(7)
    wait_out(0)
    wait_out(1)

    @pl.when(k < 4)
    def _rem():
      off = x0 + NCH * CW
      pltpu.sync_copy(tt_hbm.at[pl.ds(et * 8, 8), pl.ds(off, ROW)],
                      bufs[0].at[:, pl.ds(0, ROW)])
      pltpu.sync_copy(bufs[0].at[:, pl.ds(0, ROW)],
                      raw_hbm.at[et, :, pl.ds(off, ROW)])

  return body


def _make_a2():
  G = 4

  @functools.partial(
      pl.kernel,
      mesh=plsc.VectorSubcoreMesh(**_MESH),
      compiler_params=_LINEAR_PARAMS,
      out_type=jax.ShapeDtypeStruct((LIN_ROWS, ROW), jnp.float32),
      scratch_types=(
          [pltpu.VMEM((4 * G, 8, ROW), jnp.float32) for _ in range(2)]
          + [pltpu.VMEM((G * EMBED, ROW), jnp.float32) for _ in range(2)]
          + [pltpu.SemaphoreType.DMA for _ in range(4)]
      ),
  )
  def body(raw_hbm, tail_hbm, lin_hbm, *sc):
    ins = sc[0:2]
    outs = sc[2:4]
    gs = sc[4:6]
    ws2 = sc[6:8]
    wid = _wid()
    nq = 61 + (wid == 0)
    c0 = jnp.where(wid == 0, 0, 248 + (wid - 1) * 244)

    iota = lax.iota(jnp.int32, 16)
    e_vecs = [iota + h * 16 for h in range(2)]
    d1s = [e & 7 for e in e_vecs]
    d0s = [(e >> 3) * G for e in e_vecs]

    def fire_in(q, s):
      c = c0 + G * q
      for et in range(4):
        pltpu.async_copy(raw_hbm.at[pl.ds(et * NTJ + c, G)],
                         ins[s].at[pl.ds(et * G, G)], gs[s])

    def wait_in(s):
      for _ in range(4):
        pltpu.make_async_copy(raw_hbm.at[pl.ds(0, G)],
                              ins[s].at[pl.ds(0, G)], gs[s]).wait()

    def transpose(s):
      def blk(i, carry):
        for j in range(256):
          half = j % 2
          v = plsc.load_gather(
              ins[s],
              [d0s[half] + i, d1s[half], jnp.full((16,), j // 2, jnp.int32)],
          )
          flat = 16 * j
          outs[s][i * EMBED + flat // 128, pl.ds(flat % 128, 16)] = v
        return carry

      lax.fori_loop(0, G, blk, 0)

    def fire_out(q, s):
      pltpu.async_copy(outs[s],
                       lin_hbm.at[pl.ds((c0 + G * q) * EMBED, G * EMBED)],
                       ws2[s])

    def wait_out(s):
      pltpu.make_async_copy(outs[s], lin_hbm.at[pl.ds(0, G * EMBED)],
                            ws2[s]).wait()

    fire_in(0, 0)

    def step(q, carry):
      for s in range(2):

        @pl.when(q % 2 == s)
        def _(s=s):
          wait_in(s)

          @pl.when(q + 1 < nq)
          def _(s=s):
            fire_in(q + 1, 1 - s)

          @pl.when(q >= 2)
          def _(s=s):
            wait_out(s)

          transpose(s)
          fire_out(q, s)

      return carry

    lax.fori_loop(0, nq, step, 0)
    wait_out(0)
    wait_out(1)

    @pl.when(wid == 17)
    def _tail():
      pltpu.sync_copy(tail_hbm, lin_hbm.at[pl.ds(NBLK * EMBED, 16)])

  return body


def _make_phase_b():
  G = 10

  @functools.partial(
      pl.kernel,
      mesh=plsc.VectorSubcoreMesh(**_MESH),
      compiler_params=_LINEAR_PARAMS,
      out_type=jax.ShapeDtypeStruct((NROWS * 4, 8, ROW), jnp.float32),
      scratch_types=(
          [pltpu.VMEM((ROWS_PER_W, ROW), jnp.int32)]
          + [pltpu.VMEM((G * ROW, EMBED), jnp.float32) for _ in range(2)]
          + [pltpu.VMEM((4, 8, ROW), jnp.float32) for _ in range(2)]
          + [pltpu.SemaphoreType.DMA for _ in range(4)]
      ),
  )
  def body(idx_hbm, tab_hbm, out_hbm, idx_all, *sc):
    rows = sc[0:2]
    trs = sc[2:4]
    gs = sc[4:6]
    ws2 = sc[6:8]
    wid = _wid()
    q0 = wid * ROWS_PER_W
    pltpu.sync_copy(idx_hbm.at[pl.ds(q0, ROWS_PER_W)], idx_all)

    iota = lax.iota(jnp.int32, 16)
    b_vecs = [iota + h * 16 for h in range(8)]

    def fire_g(t, s):
      for g in range(G):
        pltpu.async_copy(tab_hbm.at[idx_all.at[G * t + g]],
                         rows[s].at[pl.ds(g * ROW, ROW)], gs[s])

    def wait_g(s):
      pltpu.make_async_copy(tab_hbm.at[pl.ds(0, G * ROW)], rows[s],
                            gs[s]).wait()

    def transpose(g, s, p):
      for j in range(256):
        v = plsc.load_gather(
            rows[s],
            [b_vecs[j % 8] + g * ROW, jnp.full((16,), j // 8, jnp.int32)],
        )
        flat = 16 * j
        trs[p][flat // 1024, (flat % 1024) // 128, pl.ds(flat % 128, 16)] = v

    def fire_w(n, p):
      q = q0 + n
      st = q // 256
      r = q % 256
      bt = r // 8
      si = r % 8
      s = st * 8 + si
      for et in range(4):
        pltpu.async_copy(trs[p].at[et], out_hbm.at[(s * 4 + et) * 32 + bt],
                         ws2[p])

    def wait_w(p):
      pltpu.make_async_copy(trs[p], out_hbm.at[pl.ds(0, 4)], ws2[p]).wait()

    fire_g(0, 0)

    def step(t, carry):
      for s in range(2):

        @pl.when(t % 2 == s)
        def _(s=s):
          wait_g(s)

          @pl.when(t + 1 < 20)
          def _(s=s):
            fire_g(t + 1, 1 - s)

          def inner(g, carry2):
            n = G * t + g
            for p in range(2):

              @pl.when(g % 2 == p)
              def _(p=p):

                @pl.when(n >= 2)
                def _(p=p):
                  wait_w(p)

                transpose(g, s, p)
                fire_w(n, p)

            return carry2

          lax.fori_loop(0, G, inner, 0)

      return carry

    lax.fori_loop(0, 20, step, 0)
    wait_w(0)
    wait_w(1)

  return body


_a1 = _make_a1()
_a2 = _make_a2()
_phase_b = _make_phase_b()


def kernel(words_seq, table):
  raw = _a1(table.T)
  raw3 = raw.reshape(4, 8, NTJ, ROW).transpose(0, 2, 1, 3).reshape(
      4 * NTJ, 8, ROW)
  tail = table[NBLK * ROW :, :].reshape(16, ROW)
  tab = _a2(raw3, tail).reshape(VOCAB, EMBED)
  ws = words_seq.astype(jnp.int32)
  idx = ws.T.reshape(25, 8, 32, 128).transpose(0, 2, 1, 3).reshape(NROWS, ROW)
  out = _phase_b(idx, tab)
  out5 = out.reshape(SEQ, 4, 32, 8, ROW)
  return out5.transpose(2, 4, 0, 1, 3).reshape(BATCH, SEQ, EMBED)
